# baseline (speedup 1.0000x reference)
import functools

import jax
import jax.numpy as jnp
from jax import lax
from jax.experimental import pallas as pl
from jax.experimental.pallas import tpu as pltpu
from jax.experimental.pallas import tpu_sc as plsc

B, N, C, S = 32, 8192, 128, 1024
TILE = 512


def _sort_topk_body(x_ref, pos_ref, vbuf, ibuf, *, n, c, tile, s_out):
    logn = n.bit_length() - 1
    nt = n // tile

    def init_tile(ti, _):
        o = ti * tile
        tv = x_ref[0, pl.ds(o, tile), :]
        it = lax.broadcasted_iota(jnp.int32, (tile, c), 0) + o
        vbuf[0, pl.ds(o, tile), :] = tv
        vbuf[0, pl.ds(n + o, tile), :] = tv
        ibuf[0, pl.ds(o, tile), :] = it
        ibuf[0, pl.ds(n + o, tile), :] = it
        return 0

    lax.fori_loop(0, nt, init_tile, 0)

    def substage(j, k, s):
        d = lax.shift_left(1, j)
        par = lax.rem(s, 2)
        tpar = 1 - par

        def tile_body(ti, _):
            o = ti * tile
            sv = vbuf[par, pl.ds(o, tile), :]
            si = ibuf[par, pl.ds(o, tile), :]
            uv = vbuf[par, pl.ds(o + d, tile), :]
            ui = ibuf[par, pl.ds(o + d, tile), :]
            dv = vbuf[par, pl.ds(n - d + o, tile), :]
            di = ibuf[par, pl.ds(n - d + o, tile), :]
            rows = lax.broadcasted_iota(jnp.int32, (tile, 1), 0) + o
            bitb = (lax.shift_right_logical(rows, j) & 1) == 1
            asc = (lax.shift_right_logical(rows, k) & 1) == 0
            pv = jnp.where(bitb, dv, uv)
            pi = jnp.where(bitb, di, ui)
            less = (sv < pv) | ((sv == pv) & (si < pi))
            take = less == jnp.logical_xor(bitb, asc)
            nv2 = jnp.where(take, sv, pv)
            ni2 = jnp.where(take, si, pi)
            vbuf[tpar, pl.ds(o, tile), :] = nv2
            vbuf[tpar, pl.ds(n + o, tile), :] = nv2
            ibuf[tpar, pl.ds(o, tile), :] = ni2
            ibuf[tpar, pl.ds(n + o, tile), :] = ni2
            return 0

        lax.fori_loop(0, nt, tile_body, 0)
        return s + 1

    def merge(k, s):
        return lax.fori_loop(0, k, lambda t, s2: substage(k - 1 - t, k, s2), s)

    s_fin = lax.fori_loop(1, logn + 1, merge, 0)
    fpar = lax.rem(s_fin, 2)

    cparts = []
    for ti in range(nt):
        o = ti * tile
        blk = ibuf[fpar, pl.ds(o, tile), :]
        blk3 = blk.reshape(tile // 128, 128, c)
        cparts.append(jnp.sum(blk3, axis=-1))
    cnt = jnp.concatenate(cparts, axis=0)

    r = n // 128
    pos = (lax.broadcasted_iota(jnp.int32, (r, 128), 0) * 128
           + lax.broadcasted_iota(jnp.int32, (r, 128), 1))
    lanei = lax.broadcasted_iota(jnp.int32, (r, 128), 1)
    rowi = lax.broadcasted_iota(jnp.int32, (r, 128), 0)
    for k in range(1, logn + 1):
        for j in reversed(range(k)):
            d = 1 << j
            if j < 7:
                bitb = ((lanei >> j) & 1) == 1
                pc = jnp.where(bitb, jnp.roll(cnt, d, axis=1),
                               jnp.roll(cnt, -d, axis=1))
                pp = jnp.where(bitb, jnp.roll(pos, d, axis=1),
                               jnp.roll(pos, -d, axis=1))
            else:
                dr = d >> 7
                bitb = ((rowi >> (j - 7)) & 1) == 1
                pc = jnp.where(bitb, jnp.roll(cnt, dr, axis=0),
                               jnp.roll(cnt, -dr, axis=0))
                pp = jnp.where(bitb, jnp.roll(pos, dr, axis=0),
                               jnp.roll(pos, -dr, axis=0))
            if k < 7:
                ascb = ((lanei >> k) & 1) == 0
            else:
                ascb = ((rowi >> (k - 7)) & 1) == 0
            first = (cnt > pc) | ((cnt == pc) & (pos < pp))
            take = first == jnp.logical_xor(bitb, ascb)
            cnt = jnp.where(take, cnt, pc)
            pos = jnp.where(take, pos, pp)

    pos_ref[0] = pos[: s_out // 128, :]


def _tc_sort_topk(x, *, n, c, tile, s_out):
    b = x.shape[0]
    body = functools.partial(_sort_topk_body, n=n, c=c, tile=tile, s_out=s_out)
    return pl.pallas_call(
        body,
        grid=(b,),
        in_specs=[pl.BlockSpec((1, n, c), lambda i: (i, 0, 0))],
        out_specs=pl.BlockSpec((1, s_out // 128, 128), lambda i: (i, 0, 0)),
        out_shape=jax.ShapeDtypeStruct((b, s_out // 128, 128), jnp.int32),
        scratch_shapes=[
            pltpu.VMEM((2, 2 * n, c), jnp.float32),
            pltpu.VMEM((2, 2 * n, c), jnp.int32),
        ],
    )(x)


_NC, _NS = 2, 16
_FCHUNK = 256


def _sc_gather_body(f_hbm, vert_hbm, idx_hbm, fout_hbm, vout_hbm,
                    idx_v, idxg_v, fbuf, sem):
    cid = lax.axis_index("c")
    sid = lax.axis_index("s")
    bid = sid * _NC + cid

    pltpu.sync_copy(idx_hbm.at[pl.ds(bid * S, S)], idx_v)

    def gbody(t, _):
        v16 = idx_v[pl.ds(t * 16, 16)]
        idxg_v[pl.ds(t * 16, 16)] = v16 + bid * N
        return 0

    lax.fori_loop(0, S // 16, gbody, 0)

    for ch in range(S // _FCHUNK):
        isl = idxg_v.at[pl.ds(ch * _FCHUNK, _FCHUNK)]
        pltpu.async_copy(f_hbm.at[isl], fbuf, sem).wait()
        pltpu.sync_copy(fbuf, fout_hbm.at[pl.ds(bid * S + ch * _FCHUNK, _FCHUNK)])
        pltpu.async_copy(vert_hbm.at[isl], fbuf, sem).wait()
        pltpu.sync_copy(fbuf, vout_hbm.at[pl.ds(bid * S + ch * _FCHUNK, _FCHUNK)])


def _sc_gather(f_flat, vert_pad, idx_flat):
    mesh = plsc.VectorSubcoreMesh(core_axis_name="c", subcore_axis_name="s")
    fn = pl.kernel(
        _sc_gather_body,
        mesh=mesh,
        out_type=[
            jax.ShapeDtypeStruct((B * S, C), jnp.float32),
            jax.ShapeDtypeStruct((B * S, C), jnp.float32),
        ],
        scratch_types=[
            pltpu.VMEM((S,), jnp.int32),
            pltpu.VMEM((S,), jnp.int32),
            pltpu.VMEM((_FCHUNK, C), jnp.float32),
            pltpu.SemaphoreType.DMA,
        ],
    )
    return fn(f_flat, vert_pad, idx_flat)


def kernel(F0, vertices0, k):
    pos = _tc_sort_topk(F0, n=N, c=C, tile=TILE, s_out=S)
    idx = pos.reshape(B, S)
    vert_pad = jnp.pad(vertices0.reshape(B * N, 3), ((0, 0), (0, C - 3)))
    f_out, v_out = _sc_gather(F0.reshape(B * N, C), vert_pad, idx.reshape(B * S))
    return f_out.reshape(B, S, C), v_out[:, :3].reshape(B, S, 3)


# speedup vs baseline: 1.5326x; 1.5326x over previous
import functools

import jax
import jax.numpy as jnp
from jax import lax
from jax.experimental import pallas as pl
from jax.experimental.pallas import tpu as pltpu
from jax.experimental.pallas import tpu_sc as plsc

B, N, C, S = 32, 8192, 128, 1024
TILE = 512


def _cmp_less(sv, pv, si, pi):
    return (sv < pv) | ((sv == pv) & (si < pi))


def _cascade_tile(v, i, o, k, t):
    rows = lax.broadcasted_iota(jnp.int32, (t, 1), 0)
    if k <= 6:
        asc = ((rows >> k) & 1) == 0
    else:
        asc = ((o >> k) & 1) == 0
    for j in range(min(k - 1, 6), -1, -1):
        d = 1 << j
        bitb = ((rows >> j) & 1) == 1
        if d >= 4:
            c2 = v.shape[1]
            pv = jnp.roll(v.reshape(t // (2 * d), 2 * d, c2), d,
                          axis=1).reshape(t, c2)
            pi = jnp.roll(i.reshape(t // (2 * d), 2 * d, c2), d,
                          axis=1).reshape(t, c2)
        else:
            pv = jnp.where(bitb, jnp.roll(v, d, axis=0),
                           jnp.roll(v, -d, axis=0))
            pi = jnp.where(bitb, jnp.roll(i, d, axis=0),
                           jnp.roll(i, -d, axis=0))
        less = _cmp_less(v, pv, i, pi)
        take = less == jnp.logical_xor(bitb, asc)
        v = jnp.where(take, v, pv)
        i = jnp.where(take, i, pi)
    return v, i


def _sort_topk_body(x_ref, pos_ref, vbuf, ibuf, cbuf, *, n, c, s_out):
    t = 128
    nt = n // t
    logn = n.bit_length() - 1

    def pass_a(ti, _):
        o = ti * t
        v = x_ref[0, pl.ds(o, t), :]
        i = lax.broadcasted_iota(jnp.int32, (t, c), 0) + o
        for k in range(1, min(7, logn) + 1):
            v, i = _cascade_tile(v, i, o, k, t)
        vbuf[pl.ds(o, t), :] = v
        ibuf[pl.ds(o, t), :] = i
        return 0

    lax.fori_loop(0, nt, pass_a, 0)

    for k in range(8, logn + 1):
        for j in range(k - 1, 6, -1):
            d = 1 << j
            m = d // t

            def pair_pass(ti, _, k=k, d=d, m=m):
                q = ti // m
                r2 = ti - q * m
                o = (2 * q * m + r2) * t
                po = o + d
                av = vbuf[pl.ds(o, t), :]
                ai = ibuf[pl.ds(o, t), :]
                bv = vbuf[pl.ds(po, t), :]
                bi = ibuf[pl.ds(po, t), :]
                asc = ((o >> k) & 1) == 0
                less = _cmp_less(av, bv, ai, bi)
                takea = less == asc
                vbuf[pl.ds(o, t), :] = jnp.where(takea, av, bv)
                ibuf[pl.ds(o, t), :] = jnp.where(takea, ai, bi)
                vbuf[pl.ds(po, t), :] = jnp.where(takea, bv, av)
                ibuf[pl.ds(po, t), :] = jnp.where(takea, bi, ai)
                return 0

            lax.fori_loop(0, nt // 2, pair_pass, 0)

        last = k == logn

        def cascade_pass(ti, _, k=k, last=last):
            o = ti * t
            v = vbuf[pl.ds(o, t), :]
            i = ibuf[pl.ds(o, t), :]
            v, i = _cascade_tile(v, i, o, k, t)
            if last:
                red = jnp.sum(i.reshape(1, t, c), axis=-1)
                cbuf[pl.ds(ti, 1), :] = red
            else:
                vbuf[pl.ds(o, t), :] = v
                ibuf[pl.ds(o, t), :] = i
            return 0

        lax.fori_loop(0, nt, cascade_pass, 0)

    r = n // 128
    cnt = cbuf[...]
    pos = (lax.broadcasted_iota(jnp.int32, (r, 128), 0) * 128
           + lax.broadcasted_iota(jnp.int32, (r, 128), 1))
    lanei = lax.broadcasted_iota(jnp.int32, (r, 128), 1)
    rowi = lax.broadcasted_iota(jnp.int32, (r, 128), 0)
    for k in range(1, logn + 1):
        for j in reversed(range(k)):
            d = 1 << j
            if j < 7:
                bitb = ((lanei >> j) & 1) == 1
                pc = jnp.where(bitb, jnp.roll(cnt, d, axis=1),
                               jnp.roll(cnt, -d, axis=1))
                pp = jnp.where(bitb, jnp.roll(pos, d, axis=1),
                               jnp.roll(pos, -d, axis=1))
            else:
                dr = d >> 7
                bitb = ((rowi >> (j - 7)) & 1) == 1
                pc = jnp.where(bitb, jnp.roll(cnt, dr, axis=0),
                               jnp.roll(cnt, -dr, axis=0))
                pp = jnp.where(bitb, jnp.roll(pos, dr, axis=0),
                               jnp.roll(pos, -dr, axis=0))
            if k < 7:
                ascb = ((lanei >> k) & 1) == 0
            else:
                ascb = ((rowi >> (k - 7)) & 1) == 0
            first = (cnt > pc) | ((cnt == pc) & (pos < pp))
            take = first == jnp.logical_xor(bitb, ascb)
            cnt = jnp.where(take, cnt, pc)
            pos = jnp.where(take, pos, pp)

    pos_ref[0] = pos[: s_out // 128, :]


def _tc_sort_topk(x, *, n, c, s_out):
    b = x.shape[0]
    body = functools.partial(_sort_topk_body, n=n, c=c, s_out=s_out)
    return pl.pallas_call(
        body,
        grid=(b,),
        in_specs=[pl.BlockSpec((1, n, c), lambda i: (i, 0, 0))],
        out_specs=pl.BlockSpec((1, s_out // 128, 128), lambda i: (i, 0, 0)),
        out_shape=jax.ShapeDtypeStruct((b, s_out // 128, 128), jnp.int32),
        scratch_shapes=[
            pltpu.VMEM((n, c), jnp.float32),
            pltpu.VMEM((n, c), jnp.int32),
            pltpu.VMEM((n // 128, 128), jnp.int32),
        ],
    )(x)


_NC, _NS = 2, 16
_FCHUNK = 256


def _sc_gather_body(f_hbm, vert_hbm, idx_hbm, fout_hbm, vout_hbm,
                    idx_v, idxg_v, fbuf, sem):
    cid = lax.axis_index("c")
    sid = lax.axis_index("s")
    bid = sid * _NC + cid

    pltpu.sync_copy(idx_hbm.at[pl.ds(bid * S, S)], idx_v)

    def gbody(t, _):
        v16 = idx_v[pl.ds(t * 16, 16)]
        idxg_v[pl.ds(t * 16, 16)] = v16 + bid * N
        return 0

    lax.fori_loop(0, S // 16, gbody, 0)

    for ch in range(S // _FCHUNK):
        isl = idxg_v.at[pl.ds(ch * _FCHUNK, _FCHUNK)]
        pltpu.async_copy(f_hbm.at[isl], fbuf, sem).wait()
        pltpu.sync_copy(fbuf, fout_hbm.at[pl.ds(bid * S + ch * _FCHUNK, _FCHUNK)])
        pltpu.async_copy(vert_hbm.at[isl], fbuf, sem).wait()
        pltpu.sync_copy(fbuf, vout_hbm.at[pl.ds(bid * S + ch * _FCHUNK, _FCHUNK)])


def _sc_gather(f_flat, vert_pad, idx_flat):
    mesh = plsc.VectorSubcoreMesh(core_axis_name="c", subcore_axis_name="s")
    fn = pl.kernel(
        _sc_gather_body,
        mesh=mesh,
        out_type=[
            jax.ShapeDtypeStruct((B * S, C), jnp.float32),
            jax.ShapeDtypeStruct((B * S, C), jnp.float32),
        ],
        scratch_types=[
            pltpu.VMEM((S,), jnp.int32),
            pltpu.VMEM((S,), jnp.int32),
            pltpu.VMEM((_FCHUNK, C), jnp.float32),
            pltpu.SemaphoreType.DMA,
        ],
    )
    return fn(f_flat, vert_pad, idx_flat)


def kernel(F0, vertices0, k):
    pos = _tc_sort_topk(F0, n=N, c=C, s_out=S)
    idx = pos.reshape(B, S)
    vert_pad = jnp.pad(vertices0.reshape(B * N, 3), ((0, 0), (0, C - 3)))
    f_out, v_out = _sc_gather(F0.reshape(B * N, C), vert_pad, idx.reshape(B * S))
    return f_out.reshape(B, S, C), v_out[:, :3].reshape(B, S, 3)


# speedup vs baseline: 1.8124x; 1.1826x over previous
import functools

import jax
import jax.numpy as jnp
from jax import lax
from jax.experimental import pallas as pl
from jax.experimental.pallas import tpu as pltpu
from jax.experimental.pallas import tpu_sc as plsc

B, N, C, S = 32, 8192, 128, 1024
TILE = 512


def _cmp_less(sv, pv, si, pi):
    return (sv < pv) | ((sv == pv) & (si < pi))


def _cascade_tile(v, i, o, k, t):
    c2 = v.shape[1]
    rows2 = lax.broadcasted_iota(jnp.int32, (t, c2), 0)
    if k <= 6:
        ascbit = (rows2 >> k) & 1
    else:
        ascbit = (o >> k) & 1
    for j in range(min(k - 1, 6), -1, -1):
        d = 1 << j
        bitj = (rows2 >> j) & 1
        if d >= 4:
            pv = jnp.roll(v.reshape(t // (2 * d), 2 * d, c2), d,
                          axis=1).reshape(t, c2)
            pi = jnp.roll(i.reshape(t // (2 * d), 2 * d, c2), d,
                          axis=1).reshape(t, c2)
        else:
            bitm = bitj == 1
            pv = jnp.where(bitm, jnp.roll(v, d, axis=0),
                           jnp.roll(v, -d, axis=0))
            pi = jnp.where(bitm, jnp.roll(i, d, axis=0),
                           jnp.roll(i, -d, axis=0))
        less = _cmp_less(v, pv, i, pi)
        wantm = (bitj ^ ascbit) == 0
        swap = jnp.logical_xor(less, wantm)
        v = jnp.where(swap, pv, v)
        i = jnp.where(swap, pi, i)
    return v, i


def _sort_topk_body(x_ref, pos_ref, vbuf, ibuf, cbuf, *, n, c, s_out):
    t = 128
    nt = n // t
    logn = n.bit_length() - 1

    def pass_a(ti, _):
        o = ti * t
        v = x_ref[0, pl.ds(o, t), :]
        i = lax.broadcasted_iota(jnp.int32, (t, c), 0) + o
        for k in range(1, min(7, logn) + 1):
            v, i = _cascade_tile(v, i, o, k, t)
        vbuf[pl.ds(o, t), :] = v
        ibuf[pl.ds(o, t), :] = i
        return 0

    lax.fori_loop(0, nt, pass_a, 0)

    for k in range(8, logn + 1):
        for j in range(k - 1, 6, -1):
            d = 1 << j
            m = d // t

            def pair_pass(ti, _, j=j, k=k, d=d, m=m):
                q = ti // m
                r = ti - q * m
                o = (2 * q * m + r) * t
                po = o + d
                av = vbuf[pl.ds(o, t), :]
                ai = ibuf[pl.ds(o, t), :]
                bv = vbuf[pl.ds(po, t), :]
                bi = ibuf[pl.ds(po, t), :]
                asc = ((o >> k) & 1) == 0
                less = _cmp_less(av, bv, ai, bi)
                takea = less == asc
                vbuf[pl.ds(o, t), :] = jnp.where(takea, av, bv)
                ibuf[pl.ds(o, t), :] = jnp.where(takea, ai, bi)
                vbuf[pl.ds(po, t), :] = jnp.where(takea, bv, av)
                ibuf[pl.ds(po, t), :] = jnp.where(takea, bi, ai)
                return 0

            lax.fori_loop(0, nt // 2, pair_pass, 0)

        last = k == logn

        def cascade_pass(ti, _, k=k, last=last):
            o = ti * t
            v = vbuf[pl.ds(o, t), :]
            i = ibuf[pl.ds(o, t), :]
            v, i = _cascade_tile(v, i, o, k, t)
            if last:
                red = jnp.sum(i.reshape(1, t, c), axis=-1)
                cbuf[pl.ds(ti, 1), :] = red
            else:
                vbuf[pl.ds(o, t), :] = v
                ibuf[pl.ds(o, t), :] = i
            return 0

        lax.fori_loop(0, nt, cascade_pass, 0)

    r = n // 128
    cnt = cbuf[...]
    pos = (lax.broadcasted_iota(jnp.int32, (r, 128), 0) * 128
           + lax.broadcasted_iota(jnp.int32, (r, 128), 1))
    lanei = lax.broadcasted_iota(jnp.int32, (r, 128), 1)
    rowi = lax.broadcasted_iota(jnp.int32, (r, 128), 0)
    for k in range(1, logn + 1):
        for j in reversed(range(k)):
            d = 1 << j
            if j < 7:
                bitb = ((lanei >> j) & 1) == 1
                pc = jnp.where(bitb, jnp.roll(cnt, d, axis=1),
                               jnp.roll(cnt, -d, axis=1))
                pp = jnp.where(bitb, jnp.roll(pos, d, axis=1),
                               jnp.roll(pos, -d, axis=1))
            else:
                dr = d >> 7
                bitb = ((rowi >> (j - 7)) & 1) == 1
                pc = jnp.where(bitb, jnp.roll(cnt, dr, axis=0),
                               jnp.roll(cnt, -dr, axis=0))
                pp = jnp.where(bitb, jnp.roll(pos, dr, axis=0),
                               jnp.roll(pos, -dr, axis=0))
            if k < 7:
                ascb = ((lanei >> k) & 1) == 0
            else:
                ascb = ((rowi >> (k - 7)) & 1) == 0
            first = (cnt > pc) | ((cnt == pc) & (pos < pp))
            take = first == jnp.logical_xor(bitb, ascb)
            cnt = jnp.where(take, cnt, pc)
            pos = jnp.where(take, pos, pp)

    pos_ref[0] = pos[: s_out // 128, :]


def _tc_sort_topk(x, *, n, c, s_out):
    b = x.shape[0]
    body = functools.partial(_sort_topk_body, n=n, c=c, s_out=s_out)
    return pl.pallas_call(
        body,
        grid=(b,),
        in_specs=[pl.BlockSpec((1, n, c), lambda i: (i, 0, 0))],
        out_specs=pl.BlockSpec((1, s_out // 128, 128), lambda i: (i, 0, 0)),
        out_shape=jax.ShapeDtypeStruct((b, s_out // 128, 128), jnp.int32),
        scratch_shapes=[
            pltpu.VMEM((n, c), jnp.float32),
            pltpu.VMEM((n, c), jnp.int32),
            pltpu.VMEM((n // 128, 128), jnp.int32),
        ],
    )(x)


_NC, _NS = 2, 16
_FCHUNK = 256


def _sc_gather_body(f_hbm, vert_hbm, idx_hbm, fout_hbm, vout_hbm,
                    idx_v, idxg_v, fbuf, sem):
    cid = lax.axis_index("c")
    sid = lax.axis_index("s")
    bid = sid * _NC + cid

    pltpu.sync_copy(idx_hbm.at[pl.ds(bid * S, S)], idx_v)

    def gbody(t, _):
        v16 = idx_v[pl.ds(t * 16, 16)]
        idxg_v[pl.ds(t * 16, 16)] = v16 + bid * N
        return 0

    lax.fori_loop(0, S // 16, gbody, 0)

    for ch in range(S // _FCHUNK):
        isl = idxg_v.at[pl.ds(ch * _FCHUNK, _FCHUNK)]
        pltpu.async_copy(f_hbm.at[isl], fbuf, sem).wait()
        pltpu.sync_copy(fbuf, fout_hbm.at[pl.ds(bid * S + ch * _FCHUNK, _FCHUNK)])
        pltpu.async_copy(vert_hbm.at[isl], fbuf, sem).wait()
        pltpu.sync_copy(fbuf, vout_hbm.at[pl.ds(bid * S + ch * _FCHUNK, _FCHUNK)])


def _sc_gather(f_flat, vert_pad, idx_flat):
    mesh = plsc.VectorSubcoreMesh(core_axis_name="c", subcore_axis_name="s")
    fn = pl.kernel(
        _sc_gather_body,
        mesh=mesh,
        out_type=[
            jax.ShapeDtypeStruct((B * S, C), jnp.float32),
            jax.ShapeDtypeStruct((B * S, C), jnp.float32),
        ],
        scratch_types=[
            pltpu.VMEM((S,), jnp.int32),
            pltpu.VMEM((S,), jnp.int32),
            pltpu.VMEM((_FCHUNK, C), jnp.float32),
            pltpu.SemaphoreType.DMA,
        ],
    )
    return fn(f_flat, vert_pad, idx_flat)


def kernel(F0, vertices0, k):
    pos = _tc_sort_topk(F0, n=N, c=C, s_out=S)
    idx = pos.reshape(B, S)
    vert_pad = jnp.pad(vertices0.reshape(B * N, 3), ((0, 0), (0, C - 3)))
    f_out, v_out = _sc_gather(F0.reshape(B * N, C), vert_pad, idx.reshape(B * S))
    return f_out.reshape(B, S, C), v_out[:, :3].reshape(B, S, 3)


# speedup vs baseline: 1.8632x; 1.0281x over previous
import functools

import jax
import jax.numpy as jnp
from jax import lax
from jax.experimental import pallas as pl
from jax.experimental.pallas import tpu as pltpu
from jax.experimental.pallas import tpu_sc as plsc

B, N, C, S = 32, 8192, 128, 1024
TILE = 512


def _cmp_less(sv, pv, si, pi):
    return (sv < pv) | ((sv == pv) & (si < pi))


def _cascade_tile(v, i, o, k, t):
    c2 = v.shape[1]
    rows2 = lax.broadcasted_iota(jnp.int32, (t, c2), 0)
    if k <= 6:
        ascbit = (rows2 >> k) & 1
    else:
        ascbit = (o >> k) & 1
    for j in range(min(k - 1, 6), -1, -1):
        d = 1 << j
        bitj = (rows2 >> j) & 1
        if d >= 4:
            pv = jnp.roll(v.reshape(t // (2 * d), 2 * d, c2), d,
                          axis=1).reshape(t, c2)
            pi = jnp.roll(i.reshape(t // (2 * d), 2 * d, c2), d,
                          axis=1).reshape(t, c2)
        else:
            bitm = bitj == 1
            pv = jnp.where(bitm, jnp.roll(v, d, axis=0),
                           jnp.roll(v, -d, axis=0))
            pi = jnp.where(bitm, jnp.roll(i, d, axis=0),
                           jnp.roll(i, -d, axis=0))
        less = _cmp_less(v, pv, i, pi)
        wantm = (bitj ^ ascbit) == 0
        swap = jnp.logical_xor(less, wantm)
        v = jnp.where(swap, pv, v)
        i = jnp.where(swap, pi, i)
    return v, i


def _sort_topk_body(x_ref, pos_ref, vbuf, ibuf, cbuf, *, n, c, s_out):
    t = 128
    nt = n // t
    logn = n.bit_length() - 1

    def pass_a(ti, _):
        o = ti * t
        v = x_ref[0, pl.ds(o, t), :]
        i = lax.broadcasted_iota(jnp.int32, (t, c), 0) + o
        for k in range(1, min(7, logn) + 1):
            v, i = _cascade_tile(v, i, o, k, t)
        vbuf[pl.ds(o, t), :] = v
        ibuf[pl.ds(o, t), :] = i
        return 0

    lax.fori_loop(0, nt, pass_a, 0)

    for k in range(8, logn + 1):
        for j in range(k - 1, 6, -1):
            d = 1 << j
            m = d // t

            def pair_pass(ti, _, j=j, k=k, d=d, m=m):
                q = ti // m
                r = ti - q * m
                o = (2 * q * m + r) * t
                po = o + d
                av = vbuf[pl.ds(o, t), :]
                ai = ibuf[pl.ds(o, t), :]
                bv = vbuf[pl.ds(po, t), :]
                bi = ibuf[pl.ds(po, t), :]
                desc = ((o >> k) & 1) == 1
                less = _cmp_less(av, bv, ai, bi)
                swap = jnp.logical_xor(less, desc)
                vbuf[pl.ds(o, t), :] = jnp.where(swap, av, bv)
                ibuf[pl.ds(o, t), :] = jnp.where(swap, ai, bi)
                vbuf[pl.ds(po, t), :] = jnp.where(swap, bv, av)
                ibuf[pl.ds(po, t), :] = jnp.where(swap, bi, ai)
                return 0

            lax.fori_loop(0, nt // 2, pair_pass, 0)

        last = k == logn

        def cascade_pass(ti, _, k=k, last=last):
            o = ti * t
            v = vbuf[pl.ds(o, t), :]
            i = ibuf[pl.ds(o, t), :]
            v, i = _cascade_tile(v, i, o, k, t)
            if last:
                red = jnp.sum(i.reshape(1, t, c), axis=-1)
                cbuf[pl.ds(ti, 1), :] = red
            else:
                vbuf[pl.ds(o, t), :] = v
                ibuf[pl.ds(o, t), :] = i
            return 0

        lax.fori_loop(0, nt, cascade_pass, 0)

    r = n // 128
    cnt = cbuf[...]
    pos = (lax.broadcasted_iota(jnp.int32, (r, 128), 0) * 128
           + lax.broadcasted_iota(jnp.int32, (r, 128), 1))
    lanei = lax.broadcasted_iota(jnp.int32, (r, 128), 1)
    rowi = lax.broadcasted_iota(jnp.int32, (r, 128), 0)
    for k in range(1, logn + 1):
        for j in reversed(range(k)):
            d = 1 << j
            if j < 7:
                bitb = ((lanei >> j) & 1) == 1
                pc = jnp.where(bitb, jnp.roll(cnt, d, axis=1),
                               jnp.roll(cnt, -d, axis=1))
                pp = jnp.where(bitb, jnp.roll(pos, d, axis=1),
                               jnp.roll(pos, -d, axis=1))
            else:
                dr = d >> 7
                bitb = ((rowi >> (j - 7)) & 1) == 1
                pc = jnp.where(bitb, jnp.roll(cnt, dr, axis=0),
                               jnp.roll(cnt, -dr, axis=0))
                pp = jnp.where(bitb, jnp.roll(pos, dr, axis=0),
                               jnp.roll(pos, -dr, axis=0))
            if k < 7:
                ascb = ((lanei >> k) & 1) == 0
            else:
                ascb = ((rowi >> (k - 7)) & 1) == 0
            first = (cnt > pc) | ((cnt == pc) & (pos < pp))
            take = first == jnp.logical_xor(bitb, ascb)
            cnt = jnp.where(take, cnt, pc)
            pos = jnp.where(take, pos, pp)

    pos_ref[0] = pos[: s_out // 128, :]


def _tc_sort_topk(x, *, n, c, s_out):
    b = x.shape[0]
    body = functools.partial(_sort_topk_body, n=n, c=c, s_out=s_out)
    return pl.pallas_call(
        body,
        grid=(b,),
        in_specs=[pl.BlockSpec((1, n, c), lambda i: (i, 0, 0))],
        out_specs=pl.BlockSpec((1, s_out // 128, 128), lambda i: (i, 0, 0)),
        out_shape=jax.ShapeDtypeStruct((b, s_out // 128, 128), jnp.int32),
        scratch_shapes=[
            pltpu.VMEM((n, c), jnp.float32),
            pltpu.VMEM((n, c), jnp.int32),
            pltpu.VMEM((n // 128, 128), jnp.int32),
        ],
    )(x)


_NC, _NS = 2, 16
_FCHUNK = 256


def _sc_gather_body(f_hbm, vert_hbm, idx_hbm, fout_hbm, vout_hbm,
                    idx_v, idxg_v, fbuf, sem):
    cid = lax.axis_index("c")
    sid = lax.axis_index("s")
    bid = sid * _NC + cid

    pltpu.sync_copy(idx_hbm.at[pl.ds(bid * S, S)], idx_v)

    def gbody(t, _):
        v16 = idx_v[pl.ds(t * 16, 16)]
        idxg_v[pl.ds(t * 16, 16)] = v16 + bid * N
        return 0

    lax.fori_loop(0, S // 16, gbody, 0)

    for ch in range(S // _FCHUNK):
        isl = idxg_v.at[pl.ds(ch * _FCHUNK, _FCHUNK)]
        pltpu.async_copy(f_hbm.at[isl], fbuf, sem).wait()
        pltpu.sync_copy(fbuf, fout_hbm.at[pl.ds(bid * S + ch * _FCHUNK, _FCHUNK)])
        pltpu.async_copy(vert_hbm.at[isl], fbuf, sem).wait()
        pltpu.sync_copy(fbuf, vout_hbm.at[pl.ds(bid * S + ch * _FCHUNK, _FCHUNK)])


def _sc_gather(f_flat, vert_pad, idx_flat):
    mesh = plsc.VectorSubcoreMesh(core_axis_name="c", subcore_axis_name="s")
    fn = pl.kernel(
        _sc_gather_body,
        mesh=mesh,
        out_type=[
            jax.ShapeDtypeStruct((B * S, C), jnp.float32),
            jax.ShapeDtypeStruct((B * S, C), jnp.float32),
        ],
        scratch_types=[
            pltpu.VMEM((S,), jnp.int32),
            pltpu.VMEM((S,), jnp.int32),
            pltpu.VMEM((_FCHUNK, C), jnp.float32),
            pltpu.SemaphoreType.DMA,
        ],
    )
    return fn(f_flat, vert_pad, idx_flat)


def kernel(F0, vertices0, k):
    pos = _tc_sort_topk(F0, n=N, c=C, s_out=S)
    idx = pos.reshape(B, S)
    vert_pad = jnp.pad(vertices0.reshape(B * N, 3), ((0, 0), (0, C - 3)))
    f_out, v_out = _sc_gather(F0.reshape(B * N, C), vert_pad, idx.reshape(B * S))
    return f_out.reshape(B, S, C), v_out[:, :3].reshape(B, S, 3)


# speedup vs baseline: 2.0304x; 1.0897x over previous
import functools

import jax
import jax.numpy as jnp
from jax import lax
from jax.experimental import pallas as pl
from jax.experimental.pallas import tpu as pltpu
from jax.experimental.pallas import tpu_sc as plsc

B, N, C, S = 32, 8192, 128, 1024
TILE = 512


def _cmp_less(sv, pv, si, pi):
    return (sv < pv) | ((sv == pv) & (si < pi))


_WOFF = {}
for _k in range(1, 7):
    for _j in range(_k):
        _WOFF[(_k, _j)] = len(_WOFF)
_BOFF = {_j: len(_WOFF) + _j for _j in range(7)}
_NMASK = len(_WOFF) + 7


def _cascade_tile(v, i, o, k, t, wref):
    c2 = v.shape[1]
    if k >= 7:
        ascbit = (o >> k) & 1
    for j in range(min(k - 1, 6), -1, -1):
        d = 1 << j
        if d >= 4:
            pv = jnp.roll(v.reshape(t // (2 * d), 2 * d, c2), d,
                          axis=1).reshape(t, c2)
            pi = jnp.roll(i.reshape(t // (2 * d), 2 * d, c2), d,
                          axis=1).reshape(t, c2)
        else:
            bitm = wref[pl.ds(_BOFF[j] * t, t), :] == 1
            pv = jnp.where(bitm, jnp.roll(v, d, axis=0),
                           jnp.roll(v, -d, axis=0))
            pi = jnp.where(bitm, jnp.roll(i, d, axis=0),
                           jnp.roll(i, -d, axis=0))
        less = _cmp_less(v, pv, i, pi)
        if k <= 6:
            wantm = wref[pl.ds(_WOFF[(k, j)] * t, t), :] == 0
        else:
            wantm = (wref[pl.ds(_BOFF[j] * t, t), :] ^ ascbit) == 0
        swap = jnp.logical_xor(less, wantm)
        v = jnp.where(swap, pv, v)
        i = jnp.where(swap, pi, i)
    return v, i


def _sort_topk_body(x_ref, pos_ref, vbuf, ibuf, cbuf, wbuf, *, n, c, s_out):
    t = 128
    nt = n // t
    logn = n.bit_length() - 1

    rows2 = lax.broadcasted_iota(jnp.int32, (t, c), 0)
    for (kk, jj), ix in _WOFF.items():
        wbuf[pl.ds(ix * t, t), :] = ((rows2 >> jj) & 1) ^ ((rows2 >> kk) & 1)
    for jj, ix in _BOFF.items():
        wbuf[pl.ds(ix * t, t), :] = (rows2 >> jj) & 1

    def pass_a(ti, _):
        o = ti * t
        v = x_ref[0, pl.ds(o, t), :]
        i = lax.broadcasted_iota(jnp.int32, (t, c), 0) + o
        for k in range(1, min(7, logn) + 1):
            v, i = _cascade_tile(v, i, o, k, t, wbuf)
        vbuf[pl.ds(o, t), :] = v
        ibuf[pl.ds(o, t), :] = i
        return 0

    lax.fori_loop(0, nt, pass_a, 0)

    for k in range(8, logn + 1):
        for j in range(k - 1, 6, -1):
            d = 1 << j
            m = d // t

            def pair_pass(ti, _, j=j, k=k, d=d, m=m):
                q = ti // m
                r = ti - q * m
                o = (2 * q * m + r) * t
                po = o + d
                av = vbuf[pl.ds(o, t), :]
                ai = ibuf[pl.ds(o, t), :]
                bv = vbuf[pl.ds(po, t), :]
                bi = ibuf[pl.ds(po, t), :]
                desc = ((o >> k) & 1) == 1
                less = _cmp_less(av, bv, ai, bi)
                swap = jnp.logical_xor(less, desc)
                vbuf[pl.ds(o, t), :] = jnp.where(swap, av, bv)
                ibuf[pl.ds(o, t), :] = jnp.where(swap, ai, bi)
                vbuf[pl.ds(po, t), :] = jnp.where(swap, bv, av)
                ibuf[pl.ds(po, t), :] = jnp.where(swap, bi, ai)
                return 0

            lax.fori_loop(0, nt // 2, pair_pass, 0)

        last = k == logn

        def cascade_pass(ti, _, k=k, last=last):
            o = ti * t
            v = vbuf[pl.ds(o, t), :]
            i = ibuf[pl.ds(o, t), :]
            v, i = _cascade_tile(v, i, o, k, t, wbuf)
            if last:
                red = jnp.sum(i.reshape(1, t, c), axis=-1)
                cbuf[pl.ds(ti, 1), :] = red
            else:
                vbuf[pl.ds(o, t), :] = v
                ibuf[pl.ds(o, t), :] = i
            return 0

        lax.fori_loop(0, nt, cascade_pass, 0)

    r = n // 128
    cnt = cbuf[...]
    pos = (lax.broadcasted_iota(jnp.int32, (r, 128), 0) * 128
           + lax.broadcasted_iota(jnp.int32, (r, 128), 1))
    lanei = lax.broadcasted_iota(jnp.int32, (r, 128), 1)
    rowi = lax.broadcasted_iota(jnp.int32, (r, 128), 0)
    for k in range(1, logn + 1):
        for j in reversed(range(k)):
            d = 1 << j
            if j < 7:
                bitb = ((lanei >> j) & 1) == 1
                pc = jnp.where(bitb, jnp.roll(cnt, d, axis=1),
                               jnp.roll(cnt, -d, axis=1))
                pp = jnp.where(bitb, jnp.roll(pos, d, axis=1),
                               jnp.roll(pos, -d, axis=1))
            else:
                dr = d >> 7
                bitb = ((rowi >> (j - 7)) & 1) == 1
                pc = jnp.where(bitb, jnp.roll(cnt, dr, axis=0),
                               jnp.roll(cnt, -dr, axis=0))
                pp = jnp.where(bitb, jnp.roll(pos, dr, axis=0),
                               jnp.roll(pos, -dr, axis=0))
            if k < 7:
                ascb = ((lanei >> k) & 1) == 0
            else:
                ascb = ((rowi >> (k - 7)) & 1) == 0
            first = (cnt > pc) | ((cnt == pc) & (pos < pp))
            take = first == jnp.logical_xor(bitb, ascb)
            cnt = jnp.where(take, cnt, pc)
            pos = jnp.where(take, pos, pp)

    pos_ref[0] = pos[: s_out // 128, :]


def _tc_sort_topk(x, *, n, c, s_out):
    b = x.shape[0]
    body = functools.partial(_sort_topk_body, n=n, c=c, s_out=s_out)
    return pl.pallas_call(
        body,
        grid=(b,),
        in_specs=[pl.BlockSpec((1, n, c), lambda i: (i, 0, 0))],
        out_specs=pl.BlockSpec((1, s_out // 128, 128), lambda i: (i, 0, 0)),
        out_shape=jax.ShapeDtypeStruct((b, s_out // 128, 128), jnp.int32),
        scratch_shapes=[
            pltpu.VMEM((n, c), jnp.float32),
            pltpu.VMEM((n, c), jnp.int32),
            pltpu.VMEM((n // 128, 128), jnp.int32),
            pltpu.VMEM((_NMASK * 128, c), jnp.int32),
        ],
    )(x)


_NC, _NS = 2, 16
_FCHUNK = 256


def _sc_gather_body(f_hbm, vert_hbm, idx_hbm, fout_hbm, vout_hbm,
                    idx_v, idxg_v, fbuf, sem):
    cid = lax.axis_index("c")
    sid = lax.axis_index("s")
    bid = sid * _NC + cid

    pltpu.sync_copy(idx_hbm.at[pl.ds(bid * S, S)], idx_v)

    def gbody(t, _):
        v16 = idx_v[pl.ds(t * 16, 16)]
        idxg_v[pl.ds(t * 16, 16)] = v16 + bid * N
        return 0

    lax.fori_loop(0, S // 16, gbody, 0)

    for ch in range(S // _FCHUNK):
        isl = idxg_v.at[pl.ds(ch * _FCHUNK, _FCHUNK)]
        pltpu.async_copy(f_hbm.at[isl], fbuf, sem).wait()
        pltpu.sync_copy(fbuf, fout_hbm.at[pl.ds(bid * S + ch * _FCHUNK, _FCHUNK)])
        pltpu.async_copy(vert_hbm.at[isl], fbuf, sem).wait()
        pltpu.sync_copy(fbuf, vout_hbm.at[pl.ds(bid * S + ch * _FCHUNK, _FCHUNK)])


def _sc_gather(f_flat, vert_pad, idx_flat):
    mesh = plsc.VectorSubcoreMesh(core_axis_name="c", subcore_axis_name="s")
    fn = pl.kernel(
        _sc_gather_body,
        mesh=mesh,
        out_type=[
            jax.ShapeDtypeStruct((B * S, C), jnp.float32),
            jax.ShapeDtypeStruct((B * S, C), jnp.float32),
        ],
        scratch_types=[
            pltpu.VMEM((S,), jnp.int32),
            pltpu.VMEM((S,), jnp.int32),
            pltpu.VMEM((_FCHUNK, C), jnp.float32),
            pltpu.SemaphoreType.DMA,
        ],
    )
    return fn(f_flat, vert_pad, idx_flat)


def kernel(F0, vertices0, k):
    pos = _tc_sort_topk(F0, n=N, c=C, s_out=S)
    idx = pos.reshape(B, S)
    vert_pad = jnp.pad(vertices0.reshape(B * N, 3), ((0, 0), (0, C - 3)))
    f_out, v_out = _sc_gather(F0.reshape(B * N, C), vert_pad, idx.reshape(B * S))
    return f_out.reshape(B, S, C), v_out[:, :3].reshape(B, S, 3)


# speedup vs baseline: 2.0307x; 1.0002x over previous
import functools

import jax
import jax.numpy as jnp
from jax import lax
from jax.experimental import pallas as pl
from jax.experimental.pallas import tpu as pltpu
from jax.experimental.pallas import tpu_sc as plsc

B, N, C, S = 32, 8192, 128, 1024


def _cmp_less(sv, pv, si, pi):
    return (sv < pv) | ((sv == pv) & (si < pi))


_WOFF = {}
for _k in range(1, 7):
    for _j in range(_k):
        _WOFF[(_k, _j)] = len(_WOFF)
_BOFF = {_j: len(_WOFF) + _j for _j in range(7)}
_NMASK = len(_WOFF) + 7


def _cascade_tile(v, i, o, k, t, wref):
    c2 = v.shape[1]
    if k >= 7:
        ascbit = (o >> k) & 1
    for j in range(min(k - 1, 6), -1, -1):
        d = 1 << j
        if d >= 4:
            pv = jnp.roll(v.reshape(t // (2 * d), 2 * d, c2), d,
                          axis=1).reshape(t, c2)
            pi = jnp.roll(i.reshape(t // (2 * d), 2 * d, c2), d,
                          axis=1).reshape(t, c2)
        else:
            bitm = wref[pl.ds(_BOFF[j] * t, t), :] == 1
            pv = jnp.where(bitm, jnp.roll(v, d, axis=0),
                           jnp.roll(v, -d, axis=0))
            pi = jnp.where(bitm, jnp.roll(i, d, axis=0),
                           jnp.roll(i, -d, axis=0))
        less = _cmp_less(v, pv, i, pi)
        if k <= 6:
            wantm = wref[pl.ds(_WOFF[(k, j)] * t, t), :] == 0
        else:
            wantm = (wref[pl.ds(_BOFF[j] * t, t), :] ^ ascbit) == 0
        swap = jnp.logical_xor(less, wantm)
        v = jnp.where(swap, pv, v)
        i = jnp.where(swap, pi, i)
    return v, i


def _sort_topk_body(x_ref, pos_ref, vbuf, ibuf, cbuf, wbuf, *, n, c, s_out):
    t = 128
    nt = n // t
    logn = n.bit_length() - 1

    rows2 = lax.broadcasted_iota(jnp.int32, (t, c), 0)
    for (kk, jj), ix in _WOFF.items():
        wbuf[pl.ds(ix * t, t), :] = ((rows2 >> jj) & 1) ^ ((rows2 >> kk) & 1)
    for jj, ix in _BOFF.items():
        wbuf[pl.ds(ix * t, t), :] = (rows2 >> jj) & 1

    def pass_a(ti, _):
        o = ti * t
        v = x_ref[0, pl.ds(o, t), :]
        i = lax.broadcasted_iota(jnp.int32, (t, c), 0) + o
        for k in range(1, min(7, logn) + 1):
            v, i = _cascade_tile(v, i, o, k, t, wbuf)
        vbuf[pl.ds(o, t), :] = v
        ibuf[pl.ds(o, t), :] = i
        return 0

    lax.fori_loop(0, nt, pass_a, 0)

    for k in range(8, logn + 1):
        for j in range(k - 1, 6, -1):
            d = 1 << j
            m = d // t

            def pair_pass(ti, _, j=j, k=k, d=d, m=m):
                q = ti // m
                r = ti - q * m
                o = (2 * q * m + r) * t
                po = o + d
                av = vbuf[pl.ds(o, t), :]
                ai = ibuf[pl.ds(o, t), :]
                bv = vbuf[pl.ds(po, t), :]
                bi = ibuf[pl.ds(po, t), :]
                desc = ((o >> k) & 1) == 1
                less = _cmp_less(av, bv, ai, bi)
                swap = jnp.logical_xor(less, desc)
                vbuf[pl.ds(o, t), :] = jnp.where(swap, av, bv)
                ibuf[pl.ds(o, t), :] = jnp.where(swap, ai, bi)
                vbuf[pl.ds(po, t), :] = jnp.where(swap, bv, av)
                ibuf[pl.ds(po, t), :] = jnp.where(swap, bi, ai)
                return 0

            lax.fori_loop(0, nt // 2, pair_pass, 0)

        last = k == logn

        def cascade_pass(ti, _, k=k, last=last):
            o = ti * t
            v = vbuf[pl.ds(o, t), :]
            i = ibuf[pl.ds(o, t), :]
            v, i = _cascade_tile(v, i, o, k, t, wbuf)
            if last:
                red = jnp.sum(i.reshape(1, t, c), axis=-1)
                cbuf[pl.ds(ti, 1), :] = red
            else:
                vbuf[pl.ds(o, t), :] = v
                ibuf[pl.ds(o, t), :] = i
            return 0

        lax.fori_loop(0, nt, cascade_pass, 0)

    r = n // 128
    cnt = cbuf[...]
    pos = (lax.broadcasted_iota(jnp.int32, (r, 128), 0) * 128
           + lax.broadcasted_iota(jnp.int32, (r, 128), 1))
    lanei = lax.broadcasted_iota(jnp.int32, (r, 128), 1)
    rowi = lax.broadcasted_iota(jnp.int32, (r, 128), 0)
    for k in range(1, logn + 1):
        for j in reversed(range(k)):
            d = 1 << j
            if j < 7:
                bitb = ((lanei >> j) & 1) == 1
                pc = jnp.where(bitb, jnp.roll(cnt, d, axis=1),
                               jnp.roll(cnt, -d, axis=1))
                pp = jnp.where(bitb, jnp.roll(pos, d, axis=1),
                               jnp.roll(pos, -d, axis=1))
            else:
                dr = d >> 7
                bitb = ((rowi >> (j - 7)) & 1) == 1
                pc = jnp.where(bitb, jnp.roll(cnt, dr, axis=0),
                               jnp.roll(cnt, -dr, axis=0))
                pp = jnp.where(bitb, jnp.roll(pos, dr, axis=0),
                               jnp.roll(pos, -dr, axis=0))
            if k < 7:
                ascb = ((lanei >> k) & 1) == 0
            else:
                ascb = ((rowi >> (k - 7)) & 1) == 0
            first = (cnt > pc) | ((cnt == pc) & (pos < pp))
            take = first == jnp.logical_xor(bitb, ascb)
            cnt = jnp.where(take, cnt, pc)
            pos = jnp.where(take, pos, pp)

    pos_ref[0] = pos[: s_out // 128, :]


def _tc_sort_topk(x, *, n, c, s_out):
    b = x.shape[0]
    body = functools.partial(_sort_topk_body, n=n, c=c, s_out=s_out)
    return pl.pallas_call(
        body,
        grid=(b,),
        in_specs=[pl.BlockSpec((1, n, c), lambda i: (i, 0, 0))],
        out_specs=pl.BlockSpec((1, s_out // 128, 128), lambda i: (i, 0, 0)),
        out_shape=jax.ShapeDtypeStruct((b, s_out // 128, 128), jnp.int32),
        scratch_shapes=[
            pltpu.VMEM((n, c), jnp.float32),
            pltpu.VMEM((n, c), jnp.int32),
            pltpu.VMEM((n // 128, 128), jnp.int32),
            pltpu.VMEM((_NMASK * 128, c), jnp.int32),
        ],
    )(x)


_NC, _NS = 2, 16
_FCHUNK = 256


def _sc_gather_body(f_hbm, vert_hbm, idx_hbm, fout_hbm, vout_hbm,
                    idx_v, idxg_v, fbuf, sem):
    cid = lax.axis_index("c")
    sid = lax.axis_index("s")
    bid = sid * _NC + cid

    pltpu.sync_copy(idx_hbm.at[pl.ds(bid * S, S)], idx_v)

    def gbody(t, _):
        v16 = idx_v[pl.ds(t * 16, 16)]
        idxg_v[pl.ds(t * 16, 16)] = v16 + bid * N
        return 0

    lax.fori_loop(0, S // 16, gbody, 0)

    for ch in range(S // _FCHUNK):
        isl = idxg_v.at[pl.ds(ch * _FCHUNK, _FCHUNK)]
        pltpu.async_copy(f_hbm.at[isl], fbuf, sem).wait()
        pltpu.sync_copy(fbuf, fout_hbm.at[pl.ds(bid * S + ch * _FCHUNK, _FCHUNK)])
        pltpu.async_copy(vert_hbm.at[isl], fbuf, sem).wait()
        pltpu.sync_copy(fbuf, vout_hbm.at[pl.ds(bid * S + ch * _FCHUNK, _FCHUNK)])


def _sc_gather(f_flat, vert_pad, idx_flat):
    mesh = plsc.VectorSubcoreMesh(core_axis_name="c", subcore_axis_name="s")
    fn = pl.kernel(
        _sc_gather_body,
        mesh=mesh,
        out_type=[
            jax.ShapeDtypeStruct((B * S, C), jnp.float32),
            jax.ShapeDtypeStruct((B * S, C), jnp.float32),
        ],
        scratch_types=[
            pltpu.VMEM((S,), jnp.int32),
            pltpu.VMEM((S,), jnp.int32),
            pltpu.VMEM((_FCHUNK, C), jnp.float32),
            pltpu.SemaphoreType.DMA,
        ],
    )
    return fn(f_flat, vert_pad, idx_flat)


def kernel(F0, vertices0, k):
    pos = _tc_sort_topk(F0, n=N, c=C, s_out=S)
    idx = pos.reshape(B, S)
    vert_pad = jnp.pad(vertices0.reshape(B * N, 3), ((0, 0), (0, C - 3)))
    f_out, v_out = _sc_gather(F0.reshape(B * N, C), vert_pad, idx.reshape(B * S))
    return f_out.reshape(B, S, C), v_out[:, :3].reshape(B, S, 3)
